# Initial kernel scaffold; baseline (speedup 1.0000x reference)
import jax
import jax.numpy as jnp
from jax.experimental import pallas as pl


def kernel(inputs, tables, fm_w, W1, b1, W2, b2, W3, b3, W4, b4):
    raise NotImplementedError("write your pallas kernel here")


# speedup vs baseline: 1.2677x; 1.2677x over previous
import functools

import jax
import jax.numpy as jnp
from jax import lax
from jax.experimental import pallas as pl
from jax.experimental.pallas import tpu as pltpu
from jax.experimental.pallas import tpu_sc as plsc

_B = 16384
_F = 26
_V = 100000
_D = 16
_NW = 32
_IDX = _B * _F
_RPW = _IDX // 128 // _NW
_G = 8
_S = _RPW // _G


def _sc_gather(shifted2d, rid2d, tables2d, fm2):
    nrows = _IDX // 128
    mesh = plsc.VectorSubcoreMesh(core_axis_name="c", subcore_axis_name="s")

    @functools.partial(
        pl.kernel,
        mesh=mesh,
        compiler_params=pltpu.CompilerParams(
            use_tc_tiling_on_sc=False, needs_layout_passes=False),
        out_type=(
            jax.ShapeDtypeStruct((nrows, 128, _D), jnp.float32),
            jax.ShapeDtypeStruct((nrows, 128), jnp.float32),
        ),
        scratch_types=[
            pltpu.VMEM((_G, 128), jnp.int32),
            pltpu.VMEM((_G, 128), jnp.int32),
            pltpu.VMEM((_G, 128, _D), jnp.float32),
            pltpu.VMEM((_G * 128, _D), jnp.float32),
            pltpu.VMEM((_G, 128), jnp.float32),
            pltpu.SemaphoreType.DMA,
            pltpu.SemaphoreType.DMA,
        ],
    )
    def k(shifted_hbm, rid_hbm, tables_hbm, fm2_hbm, emb_out, fm_out,
          idx_v, rid_v, rows_v, fmrows_v, fmval_v, sem_e, sem_f):
        wid = lax.axis_index("s") * 2 + lax.axis_index("c")
        row0 = wid * _RPW

        def step(t, carry):
            r = row0 + t * _G
            pltpu.sync_copy(shifted_hbm.at[pl.ds(r, _G)], idx_v)
            pltpu.sync_copy(rid_hbm.at[pl.ds(r, _G)], rid_v)
            copies = []
            for j in range(_G):
                copies.append(pltpu.async_copy(
                    tables_hbm.at[idx_v.at[j]], rows_v.at[j], sem_e))
                copies.append(pltpu.async_copy(
                    fm2_hbm.at[rid_v.at[j]],
                    fmrows_v.at[pl.ds(j * 128, 128)], sem_f))
            for c in copies:
                c.wait()
            pltpu.sync_copy(rows_v, emb_out.at[pl.ds(r, _G)])
            lanes16 = lax.iota(jnp.int32, 16)
            for j in range(_G):
                for c in range(128 // 16):
                    base = c * 16
                    lanes = idx_v[j, pl.ds(base, 16)] & 15
                    rows = lanes16 + (j * 128 + base)
                    vals = plsc.load_gather(fmrows_v, [rows, lanes])
                    fmval_v[j, pl.ds(base, 16)] = vals
            pltpu.sync_copy(fmval_v, fm_out.at[pl.ds(r, _G)])
            return carry

        lax.fori_loop(0, _S, step, 0)

    return k(shifted2d, rid2d, tables2d, fm2)


def _tc_body(x_ref, fmv_ref, w1, b1, w2, b2, w3, b3, w4, b4, a_ref, o_ref):
    x = x_ref[...]
    fmv = fmv_ref[...]
    first = jnp.sum(fmv, axis=1, keepdims=True)
    se = jnp.dot(x, a_ref[...], preferred_element_type=jnp.float32)
    second = 0.5 * (jnp.sum(se * se, axis=1, keepdims=True)
                    - jnp.sum(x * x, axis=1, keepdims=True))
    h = jnp.maximum(jnp.dot(x, w1[...], preferred_element_type=jnp.float32)
                    + b1[...], 0.0)
    h = jnp.maximum(jnp.dot(h, w2[...], preferred_element_type=jnp.float32)
                    + b2[...], 0.0)
    h = jnp.maximum(jnp.dot(h, w3[...], preferred_element_type=jnp.float32)
                    + b3[...], 0.0)
    y2 = jnp.dot(h, w4[...], preferred_element_type=jnp.float32) + b4[...]
    o_ref[...] = jax.nn.sigmoid(first + second + y2)


def _tc_mlp(x, fmv, W1, b1, W2, b2, W3, b3, W4, b4, A, interpret=False):
    BB = 512
    din = _F * _D
    full = lambda shape: pl.BlockSpec(shape, lambda i: (0, 0))
    return pl.pallas_call(
        _tc_body,
        grid=(_B // BB,),
        in_specs=[
            pl.BlockSpec((BB, din), lambda i: (i, 0)),
            pl.BlockSpec((BB, _F), lambda i: (i, 0)),
            full(W1.shape), full(b1.shape),
            full(W2.shape), full(b2.shape),
            full(W3.shape), full(b3.shape),
            full(W4.shape), full(b4.shape),
            full(A.shape),
        ],
        out_specs=pl.BlockSpec((BB, 1), lambda i: (i, 0)),
        out_shape=jax.ShapeDtypeStruct((_B, 1), jnp.float32),
        interpret=interpret,
    )(x, fmv, W1, b1, W2, b2, W3, b3, W4, b4, A)


def kernel(inputs, tables, fm_w, W1, b1, W2, b2, W3, b3, W4, b4):
    offsets = jnp.arange(_F, dtype=jnp.int32) * _V
    shifted = (inputs.astype(jnp.int32) + offsets[None, :]).reshape(
        _IDX // 128, 128)
    rid = shifted >> 4
    tables2 = tables.reshape(_F * _V, _D)
    fm2 = fm_w.reshape(_F * _V // 16, 16)
    emb, fmv = _sc_gather(shifted, rid, tables2, fm2)
    x = emb.reshape(_B, _F * _D)
    fmvals = fmv.reshape(_B, _F)
    A = jnp.tile(jnp.eye(_D, dtype=jnp.float32), (_F, 1))
    return _tc_mlp(x, fmvals,
                   W1, b1.reshape(1, -1), W2, b2.reshape(1, -1),
                   W3, b3.reshape(1, -1), W4, b4.reshape(1, -1), A)


# speedup vs baseline: 2.7999x; 2.2086x over previous
import functools

import jax
import jax.numpy as jnp
from jax import lax
from jax.experimental import pallas as pl
from jax.experimental.pallas import tpu as pltpu
from jax.experimental.pallas import tpu_sc as plsc

_B = 16384
_F = 26
_V = 100000
_D = 16
_NW = 32
_IDX = _B * _F
_RPW = _IDX // 128 // _NW
_G = 8
_S = _RPW // _G

_FG = 4
_VCHUNK = 2048
_NC = (_V + _VCHUNK - 1) // _VCHUNK
_RROWS = _FG * _NC * _VCHUNK


def _repack_body(t_ref, o_ref):
    o_ref[...] = t_ref[...].reshape(8 * _D, _VCHUNK).T


def _tc_repack(tT):
    return pl.pallas_call(
        _repack_body,
        grid=(_FG, _NC),
        in_specs=[pl.BlockSpec((8, _D, _VCHUNK), lambda g, c: (g, 0, c))],
        out_specs=pl.BlockSpec((_VCHUNK, 128), lambda g, c: (g * _NC + c, 0)),
        out_shape=jax.ShapeDtypeStruct((_RROWS, 128), jnp.float32),
    )(tT)


def _sc_gather(shifted2d, rid2d, lane2d, tables2d, fm2):
    nrows = _IDX // 128
    mesh = plsc.VectorSubcoreMesh(core_axis_name="c", subcore_axis_name="s")

    @functools.partial(
        pl.kernel,
        mesh=mesh,
        compiler_params=pltpu.CompilerParams(
            use_tc_tiling_on_sc=False, needs_layout_passes=False),
        out_type=(
            jax.ShapeDtypeStruct((nrows, 128, _D), jnp.float32),
            jax.ShapeDtypeStruct((nrows, 128), jnp.float32),
        ),
        scratch_types=[
            pltpu.VMEM((_G, 128), jnp.int32),
            pltpu.VMEM((_G, 128), jnp.int32),
            pltpu.VMEM((_G, 128), jnp.int32),
            pltpu.VMEM((_G, 128, _D), jnp.float32),
            pltpu.VMEM((_G * 128, _D), jnp.float32),
            pltpu.VMEM((_G, 128), jnp.float32),
            pltpu.SemaphoreType.DMA,
            pltpu.SemaphoreType.DMA,
        ],
    )
    def k(shifted_hbm, rid_hbm, lane_hbm, tables_hbm, fm2_hbm, emb_out,
          fm_out, idx_v, rid_v, lane_v, rows_v, fmrows_v, fmval_v,
          sem_e, sem_f):
        wid = lax.axis_index("s") * 2 + lax.axis_index("c")
        row0 = wid * _RPW

        def step(t, carry):
            r = row0 + t * _G
            pltpu.sync_copy(shifted_hbm.at[pl.ds(r, _G)], idx_v)
            pltpu.sync_copy(rid_hbm.at[pl.ds(r, _G)], rid_v)
            pltpu.sync_copy(lane_hbm.at[pl.ds(r, _G)], lane_v)
            copies = []
            for j in range(_G):
                copies.append(pltpu.async_copy(
                    tables_hbm.at[idx_v.at[j]], rows_v.at[j], sem_e))
                copies.append(pltpu.async_copy(
                    fm2_hbm.at[rid_v.at[j]],
                    fmrows_v.at[pl.ds(j * 128, 128)], sem_f))
            for c in copies:
                c.wait()
            pltpu.sync_copy(rows_v, emb_out.at[pl.ds(r, _G)])
            lanes16 = lax.iota(jnp.int32, 16)
            for j in range(_G):
                for c in range(128 // 16):
                    base = c * 16
                    lanes = lane_v[j, pl.ds(base, 16)]
                    rows = lanes16 + (j * 128 + base)
                    vals = plsc.load_gather(fmrows_v, [rows, lanes])
                    fmval_v[j, pl.ds(base, 16)] = vals
            pltpu.sync_copy(fmval_v, fm_out.at[pl.ds(r, _G)])
            return carry

        lax.fori_loop(0, _S, step, 0)

    return k(shifted2d, rid2d, lane2d, tables2d, fm2)


def _tc_body(x_ref, fmv_ref, w1, b1, w2, b2, w3, b3, w4, b4, a_ref, o_ref):
    x = x_ref[...]
    fmv = fmv_ref[...]
    first = jnp.sum(fmv, axis=1, keepdims=True)
    se = jnp.dot(x, a_ref[...], preferred_element_type=jnp.float32)
    second = 0.5 * (jnp.sum(se * se, axis=1, keepdims=True)
                    - jnp.sum(x * x, axis=1, keepdims=True))
    h = jnp.maximum(jnp.dot(x, w1[...], preferred_element_type=jnp.float32)
                    + b1[...], 0.0)
    h = jnp.maximum(jnp.dot(h, w2[...], preferred_element_type=jnp.float32)
                    + b2[...], 0.0)
    h = jnp.maximum(jnp.dot(h, w3[...], preferred_element_type=jnp.float32)
                    + b3[...], 0.0)
    y2 = jnp.dot(h, w4[...], preferred_element_type=jnp.float32) + b4[...]
    o_ref[...] = jax.nn.sigmoid(first + second + y2)


def _tc_mlp(x, fmv, W1, b1, W2, b2, W3, b3, W4, b4, A, interpret=False):
    BB = 512
    din = _F * _D
    full = lambda shape: pl.BlockSpec(shape, lambda i: (0, 0))
    return pl.pallas_call(
        _tc_body,
        grid=(_B // BB,),
        in_specs=[
            pl.BlockSpec((BB, din), lambda i: (i, 0)),
            pl.BlockSpec((BB, _F), lambda i: (i, 0)),
            full(W1.shape), full(b1.shape),
            full(W2.shape), full(b2.shape),
            full(W3.shape), full(b3.shape),
            full(W4.shape), full(b4.shape),
            full(A.shape),
        ],
        out_specs=pl.BlockSpec((BB, 1), lambda i: (i, 0)),
        out_shape=jax.ShapeDtypeStruct((_B, 1), jnp.float32),
        interpret=interpret,
    )(x, fmv, W1, b1, W2, b2, W3, b3, W4, b4, A)


def kernel(inputs, tables, fm_w, W1, b1, W2, b2, W3, b3, W4, b4):
    f = jnp.arange(_F, dtype=jnp.int32)
    ii = inputs.astype(jnp.int32)
    tab_add = (f >> 3) * (_NC * _VCHUNK * 8) + (f & 7)
    trow = (ii * 8 + tab_add[None, :]).reshape(_IDX // 128, 128)
    glob = ii + (f * _V)[None, :]
    rid = (glob >> 4).reshape(_IDX // 128, 128)
    lane = (glob & 15).reshape(_IDX // 128, 128)
    tT = jnp.transpose(tables, (0, 2, 1))
    tables2 = _tc_repack(tT).reshape(_RROWS * 8, _D)
    fm2 = fm_w.reshape(_F * _V // 16, 16)
    emb, fmv = _sc_gather(trow, rid, lane, tables2, fm2)
    x = emb.reshape(_B, _F * _D)
    fmvals = fmv.reshape(_B, _F)
    A = jnp.tile(jnp.eye(_D, dtype=jnp.float32), (_F, 1))
    return _tc_mlp(x, fmvals,
                   W1, b1.reshape(1, -1), W2, b2.reshape(1, -1),
                   W3, b3.reshape(1, -1), W4, b4.reshape(1, -1), A)


# speedup vs baseline: 4.1578x; 1.4850x over previous
import functools

import jax
import jax.numpy as jnp
from jax import lax
from jax.experimental import pallas as pl
from jax.experimental.pallas import tpu as pltpu
from jax.experimental.pallas import tpu_sc as plsc

_B = 16384
_F = 26
_V = 100000
_D = 16
_NW = 32
_IDX = _B * _F
_RPW = _IDX // 128 // _NW
_G = 8
_S = _RPW // _G

_FG = 4
_VCHUNK = 4096
_NC = (_V + _VCHUNK - 1) // _VCHUNK
_RROWS = _FG * _NC * _VCHUNK

_FMBLK = 2048
_FMG = (_F * _V + _FMBLK * 128 - 1) // (_FMBLK * 128)
_FMROWS = _FMG * _FMBLK


def _repack_body(t_ref, o_ref):
    o_ref[...] = t_ref[...].reshape(8 * _D, _VCHUNK).T


def _tc_repack(tT):
    return pl.pallas_call(
        _repack_body,
        grid=(_FG, _NC),
        in_specs=[pl.BlockSpec((8, _D, _VCHUNK), lambda g, c: (g, 0, c))],
        out_specs=pl.BlockSpec((_VCHUNK, 128), lambda g, c: (g * _NC + c, 0)),
        out_shape=jax.ShapeDtypeStruct((_RROWS, 128), jnp.float32),
    )(tT)


def _fm_body(f_ref, o_ref):
    o_ref[...] = f_ref[...].reshape(_FMBLK, 128)


def _fm_repack(fm_row):
    return pl.pallas_call(
        _fm_body,
        grid=(_FMG,),
        in_specs=[pl.BlockSpec((1, _FMBLK * 128), lambda i: (0, i))],
        out_specs=pl.BlockSpec((_FMBLK, 128), lambda i: (i, 0)),
        out_shape=jax.ShapeDtypeStruct((_FMROWS, 128), jnp.float32),
    )(fm_row)


def _sc_gather(trow2d, glob2d, tables2d, fm2):
    nrows = _IDX // 128
    mesh = plsc.VectorSubcoreMesh(core_axis_name="c", subcore_axis_name="s")

    @functools.partial(
        pl.kernel,
        mesh=mesh,
        compiler_params=pltpu.CompilerParams(
            use_tc_tiling_on_sc=False, needs_layout_passes=False),
        out_type=(
            jax.ShapeDtypeStruct((nrows, 128, _D), jnp.float32),
            jax.ShapeDtypeStruct((nrows, 128), jnp.float32),
        ),
        scratch_types=[
            pltpu.VMEM((_G, 128), jnp.int32),
            pltpu.VMEM((_G, 128), jnp.int32),
            pltpu.VMEM((_G, 128), jnp.int32),
            pltpu.VMEM((_G, 128, _D), jnp.float32),
            pltpu.VMEM((_G * 128, _D), jnp.float32),
            pltpu.VMEM((_G, 128), jnp.float32),
            pltpu.SemaphoreType.DMA,
            pltpu.SemaphoreType.DMA,
        ],
    )
    def k(trow_hbm, glob_hbm, tables_hbm, fm2_hbm, emb_out,
          fm_out, idx_v, glob_v, rid_v, rows_v, fmrows_v, fmval_v,
          sem_e, sem_f):
        wid = lax.axis_index("s") * 2 + lax.axis_index("c")
        row0 = wid * _RPW

        def step(t, carry):
            r = row0 + t * _G
            pltpu.sync_copy(trow_hbm.at[pl.ds(r, _G)], idx_v)
            pltpu.sync_copy(glob_hbm.at[pl.ds(r, _G)], glob_v)
            for j in range(_G):
                for c in range(128 // 16):
                    base = c * 16
                    rid_v[j, pl.ds(base, 16)] = (
                        glob_v[j, pl.ds(base, 16)] >> 4)
            copies = []
            for j in range(_G):
                copies.append(pltpu.async_copy(
                    tables_hbm.at[idx_v.at[j]], rows_v.at[j], sem_e))
                copies.append(pltpu.async_copy(
                    fm2_hbm.at[rid_v.at[j]],
                    fmrows_v.at[pl.ds(j * 128, 128)], sem_f))
            for c in copies:
                c.wait()
            pltpu.sync_copy(rows_v, emb_out.at[pl.ds(r, _G)])
            lanes16 = lax.iota(jnp.int32, 16)
            for j in range(_G):
                for c in range(128 // 16):
                    base = c * 16
                    lanes = glob_v[j, pl.ds(base, 16)] & 15
                    rows = lanes16 + (j * 128 + base)
                    vals = plsc.load_gather(fmrows_v, [rows, lanes])
                    fmval_v[j, pl.ds(base, 16)] = vals
            pltpu.sync_copy(fmval_v, fm_out.at[pl.ds(r, _G)])
            return carry

        lax.fori_loop(0, _S, step, 0)

    return k(trow2d, glob2d, tables2d, fm2)


def _tc_body(x_ref, fmv_ref, w1, b1, w2, b2, w3, b3, w4, b4, a_ref, o_ref):
    x = x_ref[...]
    fmv = fmv_ref[...]
    first = jnp.sum(fmv, axis=1, keepdims=True)
    se = jnp.dot(x, a_ref[...], preferred_element_type=jnp.float32)
    second = 0.5 * (jnp.sum(se * se, axis=1, keepdims=True)
                    - jnp.sum(x * x, axis=1, keepdims=True))
    h = jnp.maximum(jnp.dot(x, w1[...], preferred_element_type=jnp.float32)
                    + b1[...], 0.0)
    h = jnp.maximum(jnp.dot(h, w2[...], preferred_element_type=jnp.float32)
                    + b2[...], 0.0)
    h = jnp.maximum(jnp.dot(h, w3[...], preferred_element_type=jnp.float32)
                    + b3[...], 0.0)
    y2 = jnp.dot(h, w4[...], preferred_element_type=jnp.float32) + b4[...]
    o_ref[...] = jax.nn.sigmoid(first + second + y2)


def _tc_mlp(x, fmv, W1, b1, W2, b2, W3, b3, W4, b4, A, interpret=False):
    BB = 512
    din = _F * _D
    full = lambda shape: pl.BlockSpec(shape, lambda i: (0, 0))
    return pl.pallas_call(
        _tc_body,
        grid=(_B // BB,),
        in_specs=[
            pl.BlockSpec((BB, din), lambda i: (i, 0)),
            pl.BlockSpec((BB, _F), lambda i: (i, 0)),
            full(W1.shape), full(b1.shape),
            full(W2.shape), full(b2.shape),
            full(W3.shape), full(b3.shape),
            full(W4.shape), full(b4.shape),
            full(A.shape),
        ],
        out_specs=pl.BlockSpec((BB, 1), lambda i: (i, 0)),
        out_shape=jax.ShapeDtypeStruct((_B, 1), jnp.float32),
        interpret=interpret,
    )(x, fmv, W1, b1, W2, b2, W3, b3, W4, b4, A)


def kernel(inputs, tables, fm_w, W1, b1, W2, b2, W3, b3, W4, b4):
    f = jnp.arange(_F, dtype=jnp.int32)
    ii = inputs.astype(jnp.int32)
    tab_add = (f >> 3) * (_NC * _VCHUNK * 8) + (f & 7)
    trow = (ii * 8 + tab_add[None, :]).reshape(_IDX // 128, 128)
    glob = (ii + (f * _V)[None, :]).reshape(_IDX // 128, 128)
    tT = jnp.transpose(tables, (0, 2, 1))
    tables2 = _tc_repack(tT).reshape(_RROWS * 8, _D)
    fm2 = _fm_repack(fm_w.reshape(1, _F * _V)).reshape(_FMROWS * 8, _D)
    emb, fmv = _sc_gather(trow, glob, tables2, fm2)
    x = emb.reshape(_B, _F * _D)
    fmvals = fmv.reshape(_B, _F)
    A = jnp.tile(jnp.eye(_D, dtype=jnp.float32), (_F, 1))
    return _tc_mlp(x, fmvals,
                   W1, b1.reshape(1, -1), W2, b2.reshape(1, -1),
                   W3, b3.reshape(1, -1), W4, b4.reshape(1, -1), A)


# speedup vs baseline: 4.2024x; 1.0107x over previous
import functools

import jax
import jax.numpy as jnp
from jax import lax
from jax.experimental import pallas as pl
from jax.experimental.pallas import tpu as pltpu
from jax.experimental.pallas import tpu_sc as plsc

_B = 16384
_F = 26
_V = 100000
_D = 16
_NW = 32
_G = 8

_VCHUNK = 4096
_NC = (_V + _VCHUNK - 1) // _VCHUNK
_HROWS = 2 * _NC * _VCHUNK
_GSTRIDE = _NC * _VCHUNK * 8

_FMBLK = 2048
_FMG = (_F * _V + _FMBLK * 128 - 1) // (_FMBLK * 128)
_FMROWS = _FMG * _FMBLK


def _repack_body(t_ref, o_ref):
    o_ref[...] = t_ref[...].reshape(8 * _D, _VCHUNK).T


def _tc_repack(tT, gbase):
    return pl.pallas_call(
        _repack_body,
        grid=(2, _NC),
        in_specs=[pl.BlockSpec((8, _D, _VCHUNK),
                               lambda g, c: (g + gbase, 0, c))],
        out_specs=pl.BlockSpec((_VCHUNK, 128), lambda g, c: (g * _NC + c, 0)),
        out_shape=jax.ShapeDtypeStruct((_HROWS, 128), jnp.float32),
    )(tT)


def _fm_body(f_ref, o_ref):
    o_ref[...] = f_ref[...].reshape(_FMBLK, 128)


def _fm_repack(fm_row):
    return pl.pallas_call(
        _fm_body,
        grid=(_FMG,),
        in_specs=[pl.BlockSpec((1, _FMBLK * 128), lambda i: (0, i))],
        out_specs=pl.BlockSpec((_FMBLK, 128), lambda i: (i, 0)),
        out_shape=jax.ShapeDtypeStruct((_FMROWS, 128), jnp.float32),
    )(fm_row)


def _sc_gather(trow2d, glob2d, tables2d, fm2, nrows):
    rpw = nrows // _NW
    steps = rpw // _G
    mesh = plsc.VectorSubcoreMesh(core_axis_name="c", subcore_axis_name="s")

    @functools.partial(
        pl.kernel,
        mesh=mesh,
        compiler_params=pltpu.CompilerParams(
            use_tc_tiling_on_sc=False, needs_layout_passes=False),
        out_type=(
            jax.ShapeDtypeStruct((nrows, 128, _D), jnp.float32),
            jax.ShapeDtypeStruct((nrows, 128), jnp.float32),
        ),
        scratch_types=[
            pltpu.VMEM((_G, 128), jnp.int32),
            pltpu.VMEM((_G, 128), jnp.int32),
            pltpu.VMEM((_G, 128), jnp.int32),
            pltpu.VMEM((_G, 128, _D), jnp.float32),
            pltpu.VMEM((_G * 128, _D), jnp.float32),
            pltpu.VMEM((_G, 128), jnp.float32),
            pltpu.SemaphoreType.DMA,
            pltpu.SemaphoreType.DMA,
        ],
    )
    def k(trow_hbm, glob_hbm, tables_hbm, fm2_hbm, emb_out,
          fm_out, idx_v, glob_v, rid_v, rows_v, fmrows_v, fmval_v,
          sem_e, sem_f):
        wid = lax.axis_index("s") * 2 + lax.axis_index("c")
        row0 = wid * rpw

        def step(t, carry):
            r = row0 + t * _G
            pltpu.sync_copy(trow_hbm.at[pl.ds(r, _G)], idx_v)
            pltpu.sync_copy(glob_hbm.at[pl.ds(r, _G)], glob_v)
            for j in range(_G):
                for c in range(128 // 16):
                    base = c * 16
                    rid_v[j, pl.ds(base, 16)] = (
                        glob_v[j, pl.ds(base, 16)] >> 4)
            copies = []
            for j in range(_G):
                copies.append(pltpu.async_copy(
                    tables_hbm.at[idx_v.at[j]], rows_v.at[j], sem_e))
                copies.append(pltpu.async_copy(
                    fm2_hbm.at[rid_v.at[j]],
                    fmrows_v.at[pl.ds(j * 128, 128)], sem_f))
            for c in copies:
                c.wait()
            pltpu.sync_copy(rows_v, emb_out.at[pl.ds(r, _G)])
            lanes16 = lax.iota(jnp.int32, 16)
            for j in range(_G):
                for c in range(128 // 16):
                    base = c * 16
                    lanes = glob_v[j, pl.ds(base, 16)] & 15
                    rows = lanes16 + (j * 128 + base)
                    vals = plsc.load_gather(fmrows_v, [rows, lanes])
                    fmval_v[j, pl.ds(base, 16)] = vals
            pltpu.sync_copy(fmval_v, fm_out.at[pl.ds(r, _G)])
            return carry

        lax.fori_loop(0, steps, step, 0)

    return k(trow2d, glob2d, tables2d, fm2)


def _tc_body(xa_ref, xb_ref, fmv_ref, w1a, w1b, b1, w2, b2, w3, b3,
             w4, b4, aa_ref, ab_ref, o_ref):
    xa = xa_ref[...]
    xb = xb_ref[...]
    fmv = fmv_ref[...]
    first = jnp.sum(fmv, axis=1, keepdims=True)
    se = (jnp.dot(xa, aa_ref[...], preferred_element_type=jnp.float32)
          + jnp.dot(xb, ab_ref[...], preferred_element_type=jnp.float32))
    second = 0.5 * (jnp.sum(se * se, axis=1, keepdims=True)
                    - jnp.sum(xa * xa, axis=1, keepdims=True)
                    - jnp.sum(xb * xb, axis=1, keepdims=True))
    h = jnp.maximum(
        jnp.dot(xa, w1a[...], preferred_element_type=jnp.float32)
        + jnp.dot(xb, w1b[...], preferred_element_type=jnp.float32)
        + b1[...], 0.0)
    h = jnp.maximum(jnp.dot(h, w2[...], preferred_element_type=jnp.float32)
                    + b2[...], 0.0)
    h = jnp.maximum(jnp.dot(h, w3[...], preferred_element_type=jnp.float32)
                    + b3[...], 0.0)
    y2 = jnp.dot(h, w4[...], preferred_element_type=jnp.float32) + b4[...]
    o_ref[...] = jax.nn.sigmoid(first + second + y2)


def _tc_mlp(xa, xb, fmv, W1a, W1b, b1, W2, b2, W3, b3, W4, b4, Aa, Ab):
    BB = 512
    full = lambda a: pl.BlockSpec(a.shape, lambda i: (0,) * a.ndim)
    return pl.pallas_call(
        _tc_body,
        grid=(_B // BB,),
        in_specs=[
            pl.BlockSpec((BB, 16 * _D), lambda i: (i, 0)),
            pl.BlockSpec((BB, 10 * _D), lambda i: (i, 0)),
            pl.BlockSpec((BB, _F), lambda i: (i, 0)),
            full(W1a), full(W1b), full(b1),
            full(W2), full(b2), full(W3), full(b3),
            full(W4), full(b4), full(Aa), full(Ab),
        ],
        out_specs=pl.BlockSpec((BB, 1), lambda i: (i, 0)),
        out_shape=jax.ShapeDtypeStruct((_B, 1), jnp.float32),
    )(xa, xb, fmv, W1a, W1b, b1, W2, b2, W3, b3, W4, b4, Aa, Ab)


def kernel(inputs, tables, fm_w, W1, b1, W2, b2, W3, b3, W4, b4):
    ii = inputs.astype(jnp.int32)
    tT = jnp.transpose(tables, (0, 2, 1))
    fm2 = _fm_repack(fm_w.reshape(1, _F * _V)).reshape(_FMROWS * 8, _D)

    halves = []
    for base, nf in ((0, 16), (16, 10)):
        fl = jnp.arange(nf, dtype=jnp.int32)
        iih = ii[:, base:base + nf]
        nrows = _B * nf // 128
        tab_add = (fl >> 3) * _GSTRIDE + (fl & 7)
        trow = (iih * 8 + tab_add[None, :]).reshape(nrows, 128)
        glob = (iih + ((fl + base) * _V)[None, :]).reshape(nrows, 128)
        tab2 = _tc_repack(tT, base // 8).reshape(_HROWS * 8, _D)
        emb, fmv = _sc_gather(trow, glob, tab2, fm2, nrows)
        halves.append((emb.reshape(_B, nf * _D), fmv.reshape(_B, nf)))

    xa, fmva = halves[0]
    xb, fmvb = halves[1]
    fmv = jnp.concatenate([fmva, fmvb], axis=1)
    eye = jnp.eye(_D, dtype=jnp.float32)
    return _tc_mlp(xa, xb, fmv,
                   W1[:16 * _D], W1[16 * _D:], b1.reshape(1, -1),
                   W2, b2.reshape(1, -1), W3, b3.reshape(1, -1),
                   W4, b4.reshape(1, -1),
                   jnp.tile(eye, (16, 1)), jnp.tile(eye, (10, 1)))


# speedup vs baseline: 4.3993x; 1.0469x over previous
import functools

import jax
import jax.numpy as jnp
from jax import lax
from jax.experimental import pallas as pl
from jax.experimental.pallas import tpu as pltpu
from jax.experimental.pallas import tpu_sc as plsc

_B = 16384
_F = 26
_V = 100000
_D = 16
_NW = 32
_G = 8

_VCHUNK = 4096
_NC = (_V + _VCHUNK - 1) // _VCHUNK
_HROWS = 2 * _NC * _VCHUNK
_GSTRIDE = _NC * _VCHUNK * 8

_FMBLK = 2048
_FMG = (_F * _V + _FMBLK * 128 - 1) // (_FMBLK * 128)
_FMROWS = _FMG * _FMBLK


def _repack_body(t_ref, o_ref):
    o_ref[...] = t_ref[...].reshape(8 * _D, _VCHUNK).T


def _tc_repack(tT, gbase):
    return pl.pallas_call(
        _repack_body,
        grid=(2, _NC),
        in_specs=[pl.BlockSpec((8, _D, _VCHUNK),
                               lambda g, c: (g + gbase, 0, c))],
        out_specs=pl.BlockSpec((_VCHUNK, 128), lambda g, c: (g * _NC + c, 0)),
        out_shape=jax.ShapeDtypeStruct((_HROWS, 128), jnp.float32),
    )(tT)


def _fm_body(f_ref, o_ref):
    o_ref[...] = f_ref[...].reshape(_FMBLK, 128)


def _fm_repack(fm_row):
    return pl.pallas_call(
        _fm_body,
        grid=(_FMG,),
        in_specs=[pl.BlockSpec((1, _FMBLK * 128), lambda i: (0, i))],
        out_specs=pl.BlockSpec((_FMBLK, 128), lambda i: (i, 0)),
        out_shape=jax.ShapeDtypeStruct((_FMROWS, 128), jnp.float32),
    )(fm_row)


def _sc_gather(trow2d, glob2d, tables2d, fm2, nrows):
    rpw = nrows // _NW
    steps = rpw // _G
    mesh = plsc.VectorSubcoreMesh(core_axis_name="c", subcore_axis_name="s")

    @functools.partial(
        pl.kernel,
        mesh=mesh,
        compiler_params=pltpu.CompilerParams(
            use_tc_tiling_on_sc=False, needs_layout_passes=False),
        out_type=(
            jax.ShapeDtypeStruct((nrows, 128, _D), jnp.float32),
            jax.ShapeDtypeStruct((nrows, 128), jnp.float32),
        ),
        scratch_types=[
            pltpu.VMEM((_G, 128), jnp.int32),
            pltpu.VMEM((_G, 128), jnp.int32),
            pltpu.VMEM((_G, 128), jnp.int32),
            pltpu.VMEM((_G, 128, _D), jnp.float32),
            pltpu.VMEM((_G * 128, _D), jnp.float32),
            pltpu.VMEM((_G, 128), jnp.float32),
            pltpu.SemaphoreType.DMA,
            pltpu.SemaphoreType.DMA,
        ],
    )
    def k(trow_hbm, glob_hbm, tables_hbm, fm2_hbm, emb_out,
          fm_out, idx_v, glob_v, rid_v, rows_v, fmrows_v, fmval_v,
          sem_e, sem_f):
        wid = lax.axis_index("s") * 2 + lax.axis_index("c")
        row0 = wid * rpw

        def step(t, carry):
            r = row0 + t * _G
            pltpu.sync_copy(trow_hbm.at[pl.ds(r, _G)], idx_v)
            pltpu.sync_copy(glob_hbm.at[pl.ds(r, _G)], glob_v)
            for j in range(_G):
                for c in range(128 // 16):
                    base = c * 16
                    rid_v[j, pl.ds(base, 16)] = (
                        glob_v[j, pl.ds(base, 16)] >> 4)
            copies = []
            for j in range(_G):
                copies.append(pltpu.async_copy(
                    tables_hbm.at[idx_v.at[j]], rows_v.at[j], sem_e))
                copies.append(pltpu.async_copy(
                    fm2_hbm.at[rid_v.at[j]],
                    fmrows_v.at[pl.ds(j * 128, 128)], sem_f))
            for c in copies:
                c.wait()
            pltpu.sync_copy(rows_v, emb_out.at[pl.ds(r, _G)])
            lanes16 = lax.iota(jnp.int32, 16)
            for j in range(_G):
                for c in range(128 // 16):
                    base = c * 16
                    lanes = glob_v[j, pl.ds(base, 16)] & 15
                    rows = lanes16 + (j * 128 + base)
                    vals = plsc.load_gather(fmrows_v, [rows, lanes])
                    fmval_v[j, pl.ds(base, 16)] = vals
            pltpu.sync_copy(fmval_v, fm_out.at[pl.ds(r, _G)])
            return carry

        lax.fori_loop(0, steps, step, 0)

    return k(trow2d, glob2d, tables2d, fm2)


def _tc_body(xa_ref, xb_ref, fmv_ref, w1a, w1b, b1, w2, b2, w3, b3,
             w4, b4, aa_ref, ab_ref, o_ref):
    xa = xa_ref[...].reshape(xa_ref.shape[0] // 2, 2 * 128)
    xb = xb_ref[...]
    fmv = fmv_ref[...]
    first = jnp.sum(fmv, axis=1, keepdims=True)
    se = (jnp.dot(xa, aa_ref[...], preferred_element_type=jnp.float32)
          + jnp.dot(xb, ab_ref[...], preferred_element_type=jnp.float32))
    second = 0.5 * (jnp.sum(se * se, axis=1, keepdims=True)
                    - jnp.sum(xa * xa, axis=1, keepdims=True)
                    - jnp.sum(xb * xb, axis=1, keepdims=True))
    h = jnp.maximum(
        jnp.dot(xa, w1a[...], preferred_element_type=jnp.float32)
        + jnp.dot(xb, w1b[...], preferred_element_type=jnp.float32)
        + b1[...], 0.0)
    h = jnp.maximum(jnp.dot(h, w2[...], preferred_element_type=jnp.float32)
                    + b2[...], 0.0)
    h = jnp.maximum(jnp.dot(h, w3[...], preferred_element_type=jnp.float32)
                    + b3[...], 0.0)
    y2 = jnp.dot(h, w4[...], preferred_element_type=jnp.float32) + b4[...]
    o_ref[...] = jax.nn.sigmoid(first + second + y2)


def _tc_mlp(xa, xb, fmv, W1a, W1b, b1, W2, b2, W3, b3, W4, b4, Aa, Ab):
    BB = 512
    full = lambda a: pl.BlockSpec(a.shape, lambda i: (0,) * a.ndim)
    return pl.pallas_call(
        _tc_body,
        grid=(_B // BB,),
        in_specs=[
            pl.BlockSpec((BB * 2, 128), lambda i: (i, 0)),
            pl.BlockSpec((BB, 10 * _D), lambda i: (i, 0)),
            pl.BlockSpec((BB, _F), lambda i: (i, 0)),
            full(W1a), full(W1b), full(b1),
            full(W2), full(b2), full(W3), full(b3),
            full(W4), full(b4), full(Aa), full(Ab),
        ],
        out_specs=pl.BlockSpec((BB, 1), lambda i: (i, 0)),
        out_shape=jax.ShapeDtypeStruct((_B, 1), jnp.float32),
    )(xa, xb, fmv, W1a, W1b, b1, W2, b2, W3, b3, W4, b4, Aa, Ab)


def kernel(inputs, tables, fm_w, W1, b1, W2, b2, W3, b3, W4, b4):
    ii = inputs.astype(jnp.int32)
    tT = jnp.transpose(tables, (0, 2, 1))
    fm2 = _fm_repack(fm_w.reshape(1, _F * _V)).reshape(_FMROWS * 8, _D)

    halves = []
    for base, nf in ((0, 16), (16, 10)):
        fl = jnp.arange(nf, dtype=jnp.int32)
        iih = ii[:, base:base + nf]
        nrows = _B * nf // 128
        tab_add = (fl >> 3) * _GSTRIDE + (fl & 7)
        trow = (iih * 8 + tab_add[None, :]).reshape(nrows, 128)
        glob = (iih + ((fl + base) * _V)[None, :]).reshape(nrows, 128)
        tab2 = _tc_repack(tT, base // 8).reshape(_HROWS * 8, _D)
        emb, fmv = _sc_gather(trow, glob, tab2, fm2, nrows)
        halves.append((emb, fmv.reshape(_B, nf)))

    xa = halves[0][0].reshape(_B * 16 * _D // 128, 128)
    fmva = halves[0][1]
    xb = halves[1][0].reshape(_B, 10 * _D)
    fmvb = halves[1][1]
    fmv = jnp.concatenate([fmva, fmvb], axis=1)
    eye = jnp.eye(_D, dtype=jnp.float32)
    return _tc_mlp(xa, xb, fmv,
                   W1[:16 * _D], W1[16 * _D:], b1.reshape(1, -1),
                   W2, b2.reshape(1, -1), W3, b3.reshape(1, -1),
                   W4, b4.reshape(1, -1),
                   jnp.tile(eye, (16, 1)), jnp.tile(eye, (10, 1)))


# speedup vs baseline: 4.6370x; 1.0540x over previous
import functools

import jax
import jax.numpy as jnp
from jax import lax
from jax.experimental import pallas as pl
from jax.experimental.pallas import tpu as pltpu
from jax.experimental.pallas import tpu_sc as plsc

_B = 16384
_F = 26
_V = 100000
_D = 16
_NW = 32
_G = 8

_VCHUNK = 8192
_NC = (_V + _VCHUNK - 1) // _VCHUNK
_HROWS = 2 * _NC * _VCHUNK
_GSTRIDE = _NC * _VCHUNK * 8

_FMBLK = 2048
_FMG = (_F * _V + _FMBLK * 128 - 1) // (_FMBLK * 128)
_FMROWS = _FMG * _FMBLK


def _repack_body(t_ref, o_ref):
    o_ref[...] = t_ref[...].reshape(8 * _D, _VCHUNK).T


def _tc_repack(tT, gbase):
    return pl.pallas_call(
        _repack_body,
        grid=(2, _NC),
        in_specs=[pl.BlockSpec((8, _D, _VCHUNK),
                               lambda g, c: (g + gbase, 0, c))],
        out_specs=pl.BlockSpec((_VCHUNK, 128), lambda g, c: (g * _NC + c, 0)),
        out_shape=jax.ShapeDtypeStruct((_HROWS, 128), jnp.float32),
    )(tT)


def _fm_body(f_ref, o_ref):
    o_ref[...] = f_ref[...].reshape(_FMBLK, 128)


def _fm_repack(fm_row):
    return pl.pallas_call(
        _fm_body,
        grid=(_FMG,),
        in_specs=[pl.BlockSpec((1, _FMBLK * 128), lambda i: (0, i))],
        out_specs=pl.BlockSpec((_FMBLK, 128), lambda i: (i, 0)),
        out_shape=jax.ShapeDtypeStruct((_FMROWS, 128), jnp.float32),
    )(fm_row)


def _sc_gather(trow2d, glob2d, tables2d, fm2, nrows):
    rpw = nrows // _NW
    steps = rpw // _G
    mesh = plsc.VectorSubcoreMesh(core_axis_name="c", subcore_axis_name="s")

    @functools.partial(
        pl.kernel,
        mesh=mesh,
        compiler_params=pltpu.CompilerParams(
            use_tc_tiling_on_sc=False, needs_layout_passes=False),
        out_type=(
            jax.ShapeDtypeStruct((nrows, 128, _D), jnp.float32),
            jax.ShapeDtypeStruct((nrows, 128), jnp.float32),
        ),
        scratch_types=[
            pltpu.VMEM((_G, 128), jnp.int32),
            pltpu.VMEM((_G, 128), jnp.int32),
            pltpu.VMEM((_G, 128), jnp.int32),
            pltpu.VMEM((_G, 128, _D), jnp.float32),
            pltpu.VMEM((_G * 128, _D), jnp.float32),
            pltpu.VMEM((_G, 128), jnp.float32),
            pltpu.SemaphoreType.DMA,
            pltpu.SemaphoreType.DMA,
        ],
    )
    def k(trow_hbm, glob_hbm, tables_hbm, fm2_hbm, emb_out,
          fm_out, idx_v, glob_v, rid_v, rows_v, fmrows_v, fmval_v,
          sem_e, sem_f):
        wid = lax.axis_index("s") * 2 + lax.axis_index("c")
        row0 = wid * rpw

        def step(t, carry):
            r = row0 + t * _G
            pltpu.sync_copy(trow_hbm.at[pl.ds(r, _G)], idx_v)
            pltpu.sync_copy(glob_hbm.at[pl.ds(r, _G)], glob_v)
            for j in range(_G):
                for c in range(128 // 16):
                    base = c * 16
                    rid_v[j, pl.ds(base, 16)] = (
                        glob_v[j, pl.ds(base, 16)] >> 4)
            copies = []
            for j in range(_G):
                copies.append(pltpu.async_copy(
                    tables_hbm.at[idx_v.at[j]], rows_v.at[j], sem_e))
                copies.append(pltpu.async_copy(
                    fm2_hbm.at[rid_v.at[j]],
                    fmrows_v.at[pl.ds(j * 128, 128)], sem_f))
            for c in copies:
                c.wait()
            pltpu.sync_copy(rows_v, emb_out.at[pl.ds(r, _G)])
            lanes16 = lax.iota(jnp.int32, 16)
            for j in range(_G):
                for c in range(128 // 16):
                    base = c * 16
                    lanes = glob_v[j, pl.ds(base, 16)] & 15
                    rows = lanes16 + (j * 128 + base)
                    vals = plsc.load_gather(fmrows_v, [rows, lanes])
                    fmval_v[j, pl.ds(base, 16)] = vals
            pltpu.sync_copy(fmval_v, fm_out.at[pl.ds(r, _G)])
            return carry

        lax.fori_loop(0, steps, step, 0)

    return k(trow2d, glob2d, tables2d, fm2)


def _tc_body(xa_ref, xb_ref, fmv_ref, w1a, w1b, b1, w2, b2, w3, b3,
             w4, b4, aa_ref, ab_ref, o_ref):
    xa = xa_ref[...].reshape(xa_ref.shape[0] // 2, 2 * 128)
    xb = xb_ref[...]
    fmv = fmv_ref[...]
    first = jnp.sum(fmv, axis=1, keepdims=True)
    se = (jnp.dot(xa, aa_ref[...], preferred_element_type=jnp.float32)
          + jnp.dot(xb, ab_ref[...], preferred_element_type=jnp.float32))
    second = 0.5 * (jnp.sum(se * se, axis=1, keepdims=True)
                    - jnp.sum(xa * xa, axis=1, keepdims=True)
                    - jnp.sum(xb * xb, axis=1, keepdims=True))
    h = jnp.maximum(
        jnp.dot(xa, w1a[...], preferred_element_type=jnp.float32)
        + jnp.dot(xb, w1b[...], preferred_element_type=jnp.float32)
        + b1[...], 0.0)
    h = jnp.maximum(jnp.dot(h, w2[...], preferred_element_type=jnp.float32)
                    + b2[...], 0.0)
    h = jnp.maximum(jnp.dot(h, w3[...], preferred_element_type=jnp.float32)
                    + b3[...], 0.0)
    y2 = jnp.dot(h, w4[...], preferred_element_type=jnp.float32) + b4[...]
    o_ref[...] = jax.nn.sigmoid(first + second + y2)


def _tc_mlp(xa, xb, fmv, W1a, W1b, b1, W2, b2, W3, b3, W4, b4, Aa, Ab):
    BB = 512
    full = lambda a: pl.BlockSpec(a.shape, lambda i: (0,) * a.ndim)
    return pl.pallas_call(
        _tc_body,
        grid=(_B // BB,),
        in_specs=[
            pl.BlockSpec((BB * 2, 128), lambda i: (i, 0)),
            pl.BlockSpec((BB, 10 * _D), lambda i: (i, 0)),
            pl.BlockSpec((BB, _F), lambda i: (i, 0)),
            full(W1a), full(W1b), full(b1),
            full(W2), full(b2), full(W3), full(b3),
            full(W4), full(b4), full(Aa), full(Ab),
        ],
        out_specs=pl.BlockSpec((BB, 1), lambda i: (i, 0)),
        out_shape=jax.ShapeDtypeStruct((_B, 1), jnp.float32),
    )(xa, xb, fmv, W1a, W1b, b1, W2, b2, W3, b3, W4, b4, Aa, Ab)


def kernel(inputs, tables, fm_w, W1, b1, W2, b2, W3, b3, W4, b4):
    ii = inputs.astype(jnp.int32)
    tT = jnp.transpose(tables, (0, 2, 1))
    fm2 = _fm_repack(fm_w.reshape(1, _F * _V)).reshape(_FMROWS * 8, _D)

    halves = []
    for base, nf in ((0, 16), (16, 10)):
        fl = jnp.arange(nf, dtype=jnp.int32)
        iih = ii[:, base:base + nf]
        nrows = _B * nf // 128
        tab_add = (fl >> 3) * _GSTRIDE + (fl & 7)
        trow = (iih * 8 + tab_add[None, :]).reshape(nrows, 128)
        glob = (iih + ((fl + base) * _V)[None, :]).reshape(nrows, 128)
        tab2 = _tc_repack(tT, base // 8).reshape(_HROWS * 8, _D)
        emb, fmv = _sc_gather(trow, glob, tab2, fm2, nrows)
        halves.append((emb, fmv.reshape(_B, nf)))

    xa = halves[0][0].reshape(_B * 16 * _D // 128, 128)
    fmva = halves[0][1]
    xb = halves[1][0].reshape(_B, 10 * _D)
    fmvb = halves[1][1]
    fmv = jnp.concatenate([fmva, fmvb], axis=1)
    eye = jnp.eye(_D, dtype=jnp.float32)
    return _tc_mlp(xa, xb, fmv,
                   W1[:16 * _D], W1[16 * _D:], b1.reshape(1, -1),
                   W2, b2.reshape(1, -1), W3, b3.reshape(1, -1),
                   W4, b4.reshape(1, -1),
                   jnp.tile(eye, (16, 1)), jnp.tile(eye, (10, 1)))


# speedup vs baseline: 4.7943x; 1.0339x over previous
import functools

import jax
import jax.numpy as jnp
from jax import lax
from jax.experimental import pallas as pl
from jax.experimental.pallas import tpu as pltpu
from jax.experimental.pallas import tpu_sc as plsc

_B = 16384
_F = 26
_V = 100000
_D = 16
_NW = 32
_G = 8

_VCHUNK = 8192
_NC = (_V + _VCHUNK - 1) // _VCHUNK
_HROWS = 2 * _NC * _VCHUNK
_GSTRIDE = _NC * _VCHUNK * 8

_FMBLK = 2048
_FMG = (_F * _V + _FMBLK * 128 - 1) // (_FMBLK * 128)
_FMROWS = _FMG * _FMBLK


def _repack_body(t_ref, o_ref):
    o_ref[...] = t_ref[...].reshape(8 * _D, _VCHUNK).T


def _tc_repack(tT, gbase):
    return pl.pallas_call(
        _repack_body,
        grid=(2, _NC),
        in_specs=[pl.BlockSpec((8, _D, _VCHUNK),
                               lambda g, c: (g + gbase, 0, c))],
        out_specs=pl.BlockSpec((_VCHUNK, 128), lambda g, c: (g * _NC + c, 0)),
        out_shape=jax.ShapeDtypeStruct((_HROWS, 128), jnp.float32),
    )(tT)


def _fm_body(f_ref, o_ref):
    o_ref[...] = f_ref[...].reshape(_FMBLK, 128)


def _fm_repack(fm_row):
    return pl.pallas_call(
        _fm_body,
        grid=(_FMG,),
        in_specs=[pl.BlockSpec((1, _FMBLK * 128), lambda i: (0, i))],
        out_specs=pl.BlockSpec((_FMBLK, 128), lambda i: (i, 0)),
        out_shape=jax.ShapeDtypeStruct((_FMROWS, 128), jnp.float32),
    )(fm_row)


def _sc_gather(trow2d, glob2d, tables2d, fm2, nrows):
    rpw = nrows // _NW
    steps = rpw // _G
    mesh = plsc.VectorSubcoreMesh(core_axis_name="c", subcore_axis_name="s")

    @functools.partial(
        pl.kernel,
        mesh=mesh,
        compiler_params=pltpu.CompilerParams(
            use_tc_tiling_on_sc=False, needs_layout_passes=False),
        out_type=(
            jax.ShapeDtypeStruct((nrows, 128, _D), jnp.float32),
            jax.ShapeDtypeStruct((nrows, 128), jnp.float32),
        ),
        scratch_types=[
            pltpu.VMEM((_G, 128), jnp.int32),
            pltpu.VMEM((_G, 128), jnp.int32),
            pltpu.VMEM((_G, 128), jnp.int32),
            pltpu.VMEM((_G, 128, _D), jnp.float32),
            pltpu.VMEM((_G * 128, _D), jnp.float32),
            pltpu.VMEM((_G, 128), jnp.float32),
            pltpu.SemaphoreType.DMA,
            pltpu.SemaphoreType.DMA,
        ],
    )
    def k(trow_hbm, glob_hbm, tables_hbm, fm2_hbm, emb_out,
          fm_out, idx_v, glob_v, rid_v, rows_v, fmrows_v, fmval_v,
          sem_e, sem_f):
        wid = lax.axis_index("s") * 2 + lax.axis_index("c")
        row0 = wid * rpw

        def step(t, carry):
            r = row0 + t * _G
            pltpu.sync_copy(trow_hbm.at[pl.ds(r, _G)], idx_v)
            pltpu.sync_copy(glob_hbm.at[pl.ds(r, _G)], glob_v)
            for j in range(_G):
                for c in range(128 // 16):
                    base = c * 16
                    rid_v[j, pl.ds(base, 16)] = (
                        glob_v[j, pl.ds(base, 16)] >> 4)
            copies = []
            for j in range(_G):
                copies.append(pltpu.async_copy(
                    tables_hbm.at[idx_v.at[j]], rows_v.at[j], sem_e))
                copies.append(pltpu.async_copy(
                    fm2_hbm.at[rid_v.at[j]],
                    fmrows_v.at[pl.ds(j * 128, 128)], sem_f))
            for c in copies:
                c.wait()
            pltpu.sync_copy(rows_v, emb_out.at[pl.ds(r, _G)])
            lanes16 = lax.iota(jnp.int32, 16)
            for j in range(_G):
                for c in range(128 // 16):
                    base = c * 16
                    lanes = glob_v[j, pl.ds(base, 16)] & 15
                    rows = lanes16 + (j * 128 + base)
                    vals = plsc.load_gather(fmrows_v, [rows, lanes])
                    fmval_v[j, pl.ds(base, 16)] = vals
            pltpu.sync_copy(fmval_v, fm_out.at[pl.ds(r, _G)])
            return carry

        lax.fori_loop(0, steps, step, 0)

    return k(trow2d, glob2d, tables2d, fm2)


def _tc_body(xa_ref, xb_ref, fmv_ref, w1a, w1b, b1, w2, b2, w3, b3,
             w4, b4, aa_ref, ab_ref, o_ref):
    xa = xa_ref[...].reshape(xa_ref.shape[0] // 2, 2 * 128)
    xb = xb_ref[...]
    fmv = fmv_ref[...]
    first = jnp.sum(fmv, axis=1, keepdims=True)
    se = (jnp.dot(xa, aa_ref[...], preferred_element_type=jnp.float32)
          + jnp.dot(xb, ab_ref[...], preferred_element_type=jnp.float32))
    second = 0.5 * (jnp.sum(se * se, axis=1, keepdims=True)
                    - jnp.sum(xa * xa, axis=1, keepdims=True)
                    - jnp.sum(xb * xb, axis=1, keepdims=True))
    h = jnp.maximum(
        jnp.dot(xa, w1a[...], preferred_element_type=jnp.float32)
        + jnp.dot(xb, w1b[...], preferred_element_type=jnp.float32)
        + b1[...], 0.0)
    h = jnp.maximum(jnp.dot(h, w2[...], preferred_element_type=jnp.float32)
                    + b2[...], 0.0)
    h = jnp.maximum(jnp.dot(h, w3[...], preferred_element_type=jnp.float32)
                    + b3[...], 0.0)
    y2 = jnp.dot(h, w4[...], preferred_element_type=jnp.float32) + b4[...]
    o_ref[...] = jax.nn.sigmoid(first + second + y2)


def _tc_mlp(xa, xb, fmv, W1a, W1b, b1, W2, b2, W3, b3, W4, b4, Aa, Ab):
    BB = 1024
    full = lambda a: pl.BlockSpec(a.shape, lambda i: (0,) * a.ndim)
    return pl.pallas_call(
        _tc_body,
        grid=(_B // BB,),
        in_specs=[
            pl.BlockSpec((BB * 2, 128), lambda i: (i, 0)),
            pl.BlockSpec((BB, 10 * _D), lambda i: (i, 0)),
            pl.BlockSpec((BB, _F), lambda i: (i, 0)),
            full(W1a), full(W1b), full(b1),
            full(W2), full(b2), full(W3), full(b3),
            full(W4), full(b4), full(Aa), full(Ab),
        ],
        out_specs=pl.BlockSpec((BB, 1), lambda i: (i, 0)),
        out_shape=jax.ShapeDtypeStruct((_B, 1), jnp.float32),
    )(xa, xb, fmv, W1a, W1b, b1, W2, b2, W3, b3, W4, b4, Aa, Ab)


def kernel(inputs, tables, fm_w, W1, b1, W2, b2, W3, b3, W4, b4):
    ii = inputs.astype(jnp.int32)
    tT = jnp.transpose(tables, (0, 2, 1))
    fm2 = _fm_repack(fm_w.reshape(1, _F * _V)).reshape(_FMROWS * 8, _D)

    halves = []
    for base, nf in ((0, 16), (16, 10)):
        fl = jnp.arange(nf, dtype=jnp.int32)
        iih = ii[:, base:base + nf]
        nrows = _B * nf // 128
        tab_add = (fl >> 3) * _GSTRIDE + (fl & 7)
        trow = (iih * 8 + tab_add[None, :]).reshape(nrows, 128)
        glob = (iih + ((fl + base) * _V)[None, :]).reshape(nrows, 128)
        tab2 = _tc_repack(tT, base // 8).reshape(_HROWS * 8, _D)
        emb, fmv = _sc_gather(trow, glob, tab2, fm2, nrows)
        halves.append((emb, fmv.reshape(_B, nf)))

    xa = halves[0][0].reshape(_B * 16 * _D // 128, 128)
    fmva = halves[0][1]
    xb = halves[1][0].reshape(_B, 10 * _D)
    fmvb = halves[1][1]
    fmv = jnp.concatenate([fmva, fmvb], axis=1)
    eye = jnp.eye(_D, dtype=jnp.float32)
    return _tc_mlp(xa, xb, fmv,
                   W1[:16 * _D], W1[16 * _D:], b1.reshape(1, -1),
                   W2, b2.reshape(1, -1), W3, b3.reshape(1, -1),
                   W4, b4.reshape(1, -1),
                   jnp.tile(eye, (16, 1)), jnp.tile(eye, (10, 1)))


# speedup vs baseline: 4.8562x; 1.0129x over previous
import functools

import jax
import jax.numpy as jnp
from jax import lax
from jax.experimental import pallas as pl
from jax.experimental.pallas import tpu as pltpu
from jax.experimental.pallas import tpu_sc as plsc

_B = 16384
_F = 26
_V = 100000
_D = 16
_NW = 32
_G = 8

_VCHUNK = 8192
_NC = (_V + _VCHUNK - 1) // _VCHUNK
_HROWS = 2 * _NC * _VCHUNK
_GSTRIDE = _NC * _VCHUNK * 8

_FMBLK = 2048
_FMG = (_F * _V + _FMBLK * 128 - 1) // (_FMBLK * 128)
_FMROWS = _FMG * _FMBLK


def _repack_body(t_ref, o_ref):
    o_ref[...] = t_ref[...].reshape(8 * _D, _VCHUNK).T


def _tc_repack(tT, gbase):
    return pl.pallas_call(
        _repack_body,
        grid=(2, _NC),
        in_specs=[pl.BlockSpec((8, _D, _VCHUNK),
                               lambda g, c: (g + gbase, 0, c))],
        out_specs=pl.BlockSpec((_VCHUNK, 128), lambda g, c: (g * _NC + c, 0)),
        out_shape=jax.ShapeDtypeStruct((_HROWS, 128), jnp.float32),
    )(tT)


def _fm_body(f_ref, o_ref):
    o_ref[...] = f_ref[...].reshape(_FMBLK, 128)


def _fm_repack(fm_row):
    return pl.pallas_call(
        _fm_body,
        grid=(_FMG,),
        in_specs=[pl.BlockSpec((1, _FMBLK * 128), lambda i: (0, i))],
        out_specs=pl.BlockSpec((_FMBLK, 128), lambda i: (i, 0)),
        out_shape=jax.ShapeDtypeStruct((_FMROWS, 128), jnp.float32),
    )(fm_row)


def _sc_gather(trow2d, glob2d, tables2d, fm2, nrows):
    rpw = nrows // _NW
    steps = rpw // _G
    mesh = plsc.VectorSubcoreMesh(core_axis_name="c", subcore_axis_name="s")

    @functools.partial(
        pl.kernel,
        mesh=mesh,
        compiler_params=pltpu.CompilerParams(
            use_tc_tiling_on_sc=False, needs_layout_passes=False),
        out_type=(
            jax.ShapeDtypeStruct((nrows, 128, _D), jnp.float32),
            jax.ShapeDtypeStruct((nrows, 128), jnp.float32),
        ),
        scratch_types=[
            pltpu.VMEM((_G, 128), jnp.int32),
            pltpu.VMEM((_G, 128), jnp.int32),
            pltpu.VMEM((_G, 128), jnp.int32),
            pltpu.VMEM((_G, 128, _D), jnp.float32),
            pltpu.VMEM((_G * 128, _D), jnp.float32),
            pltpu.VMEM((_G, 128), jnp.float32),
            pltpu.SemaphoreType.DMA,
            pltpu.SemaphoreType.DMA,
        ],
    )
    def k(trow_hbm, glob_hbm, tables_hbm, fm2_hbm, emb_out,
          fm_out, idx_v, glob_v, rid_v, rows_v, fmrows_v, fmval_v,
          sem_e, sem_f):
        wid = lax.axis_index("s") * 2 + lax.axis_index("c")
        row0 = wid * rpw

        def step(t, carry):
            r = row0 + t * _G
            pltpu.sync_copy(trow_hbm.at[pl.ds(r, _G)], idx_v)
            pltpu.sync_copy(glob_hbm.at[pl.ds(r, _G)], glob_v)
            for j in range(_G):
                for c in range(128 // 16):
                    base = c * 16
                    rid_v[j, pl.ds(base, 16)] = (
                        glob_v[j, pl.ds(base, 16)] >> 4)
            copies = []
            for j in range(_G):
                copies.append(pltpu.async_copy(
                    tables_hbm.at[idx_v.at[j]], rows_v.at[j], sem_e))
                copies.append(pltpu.async_copy(
                    fm2_hbm.at[rid_v.at[j]],
                    fmrows_v.at[pl.ds(j * 128, 128)], sem_f))
            for c in copies:
                c.wait()
            pltpu.sync_copy(rows_v, emb_out.at[pl.ds(r, _G)])
            lanes16 = lax.iota(jnp.int32, 16)
            for j in range(_G):
                for c in range(128 // 16):
                    base = c * 16
                    lanes = glob_v[j, pl.ds(base, 16)] & 15
                    rows = lanes16 + (j * 128 + base)
                    vals = plsc.load_gather(fmrows_v, [rows, lanes])
                    fmval_v[j, pl.ds(base, 16)] = vals
            pltpu.sync_copy(fmval_v, fm_out.at[pl.ds(r, _G)])
            return carry

        lax.fori_loop(0, steps, step, 0)

    return k(trow2d, glob2d, tables2d, fm2)


def _tc_body(xa_ref, xb_ref, fmv_ref, w1a, w1b, b1, w2, b2, w3, b3,
             w4, b4, aa_ref, ab_ref, o_ref):
    xa = xa_ref[...].reshape(xa_ref.shape[0] // 2, 2 * 128)
    xb = xb_ref[...]
    fmv = fmv_ref[...]
    first = jnp.sum(fmv, axis=1, keepdims=True)
    se = (jnp.dot(xa, aa_ref[...], preferred_element_type=jnp.float32)
          + jnp.dot(xb, ab_ref[...], preferred_element_type=jnp.float32))
    second = 0.5 * (jnp.sum(se * se, axis=1, keepdims=True)
                    - jnp.sum(xa * xa, axis=1, keepdims=True)
                    - jnp.sum(xb * xb, axis=1, keepdims=True))
    h = jnp.maximum(
        jnp.dot(xa, w1a[...], preferred_element_type=jnp.float32)
        + jnp.dot(xb, w1b[...], preferred_element_type=jnp.float32)
        + b1[...], 0.0)
    h = jnp.maximum(jnp.dot(h, w2[...], preferred_element_type=jnp.float32)
                    + b2[...], 0.0)
    h = jnp.maximum(jnp.dot(h, w3[...], preferred_element_type=jnp.float32)
                    + b3[...], 0.0)
    y2 = jnp.dot(h, w4[...], preferred_element_type=jnp.float32) + b4[...]
    o_ref[...] = jax.nn.sigmoid(first + second + y2)


def _tc_mlp(xa, xb, fmv, W1a, W1b, b1, W2, b2, W3, b3, W4, b4, Aa, Ab):
    BB = 2048
    full = lambda a: pl.BlockSpec(a.shape, lambda i: (0,) * a.ndim)
    return pl.pallas_call(
        _tc_body,
        grid=(_B // BB,),
        in_specs=[
            pl.BlockSpec((BB * 2, 128), lambda i: (i, 0)),
            pl.BlockSpec((BB, 10 * _D), lambda i: (i, 0)),
            pl.BlockSpec((BB, _F), lambda i: (i, 0)),
            full(W1a), full(W1b), full(b1),
            full(W2), full(b2), full(W3), full(b3),
            full(W4), full(b4), full(Aa), full(Ab),
        ],
        out_specs=pl.BlockSpec((BB, 1), lambda i: (i, 0)),
        out_shape=jax.ShapeDtypeStruct((_B, 1), jnp.float32),
    )(xa, xb, fmv, W1a, W1b, b1, W2, b2, W3, b3, W4, b4, Aa, Ab)


def kernel(inputs, tables, fm_w, W1, b1, W2, b2, W3, b3, W4, b4):
    ii = inputs.astype(jnp.int32)
    tT = jnp.transpose(tables, (0, 2, 1))
    fm2 = _fm_repack(fm_w.reshape(1, _F * _V)).reshape(_FMROWS * 8, _D)

    halves = []
    for base, nf in ((0, 16), (16, 10)):
        fl = jnp.arange(nf, dtype=jnp.int32)
        iih = ii[:, base:base + nf]
        nrows = _B * nf // 128
        tab_add = (fl >> 3) * _GSTRIDE + (fl & 7)
        trow = (iih * 8 + tab_add[None, :]).reshape(nrows, 128)
        glob = (iih + ((fl + base) * _V)[None, :]).reshape(nrows, 128)
        tab2 = _tc_repack(tT, base // 8).reshape(_HROWS * 8, _D)
        emb, fmv = _sc_gather(trow, glob, tab2, fm2, nrows)
        halves.append((emb, fmv.reshape(_B, nf)))

    xa = halves[0][0].reshape(_B * 16 * _D // 128, 128)
    fmva = halves[0][1]
    xb = halves[1][0].reshape(_B, 10 * _D)
    fmvb = halves[1][1]
    fmv = jnp.concatenate([fmva, fmvb], axis=1)
    eye = jnp.eye(_D, dtype=jnp.float32)
    return _tc_mlp(xa, xb, fmv,
                   W1[:16 * _D], W1[16 * _D:], b1.reshape(1, -1),
                   W2, b2.reshape(1, -1), W3, b3.reshape(1, -1),
                   W4, b4.reshape(1, -1),
                   jnp.tile(eye, (16, 1)), jnp.tile(eye, (10, 1)))
